# Initial kernel scaffold; baseline (speedup 1.0000x reference)
import functools

import jax
import jax.numpy as jnp
from jax import lax
from jax.experimental import pallas as pl
from jax.experimental.pallas import tpu as pltpu
from jax.experimental.pallas import tpu_sc as plsc

N_NODES = 10000
N_EDGES = 320000
D_NODE = 128
D_EDGE = 16

NC = 2
NSUB = 16
NW = NC * NSUB
CHUNK = 128
EPW = ((N_EDGES + NW * CHUNK - 1) // (NW * CHUNK)) * CHUNK
E_PAD = NW * EPW
N_PAD = 10240
RPT = N_PAD // NSUB
ZROWS = 64


def _gine_aggregate(x, src, dst, e, d):
    mesh = plsc.VectorSubcoreMesh(core_axis_name="c", subcore_axis_name="s")

    @functools.partial(
        pl.kernel,
        out_type=jax.ShapeDtypeStruct((NC, N_PAD, d), jnp.float32),
        mesh=mesh,
        scratch_types=[
            pltpu.VMEM((CHUNK,), jnp.int32),
            pltpu.VMEM((CHUNK,), jnp.int32),
            pltpu.VMEM((CHUNK, d), jnp.float32),
            pltpu.VMEM((CHUNK, d), jnp.float32),
            pltpu.VMEM((ZROWS, d), jnp.float32),
            pltpu.VMEM_SHARED((N_PAD, d), jnp.float32),
        ],
    )
    def kern(x_hbm, src_hbm, dst_hbm, e_hbm, out_hbm,
             srcv, dstv, rows, ev, zbuf, aggr):
        cid = lax.axis_index("c")
        sid = lax.axis_index("s")
        wid = sid * NC + cid

        @pl.loop(0, ZROWS)
        def _(i):
            for j in range(d // 16):
                zbuf.at[i, pl.ds(j * 16, 16)][...] = jnp.zeros((16,), jnp.float32)

        @pl.loop(0, RPT, step=ZROWS)
        def _(r):
            pltpu.sync_copy(zbuf, aggr.at[pl.ds(sid * RPT + r, ZROWS)])

        plsc.subcore_barrier()

        @pl.loop(0, EPW, step=CHUNK)
        def _(off):
            base = wid * EPW + off
            pltpu.sync_copy(src_hbm.at[pl.ds(base, CHUNK)], srcv)
            pltpu.sync_copy(dst_hbm.at[pl.ds(base, CHUNK)], dstv)
            pltpu.sync_copy(x_hbm.at[srcv], rows)
            pltpu.sync_copy(e_hbm.at[pl.ds(base, CHUNK)], ev)

            @pl.loop(0, CHUNK)
            def _(i):
                for j in range(d // 16):
                    sl = (i, pl.ds(j * 16, 16))
                    ev.at[*sl][...] = jnp.maximum(
                        ev.at[*sl][...] + rows.at[*sl][...], 0.0)

            pltpu.sync_copy(ev, aggr.at[dstv], add=True)

        plsc.subcore_barrier()
        pltpu.sync_copy(aggr.at[pl.ds(sid * RPT, RPT)],
                        out_hbm.at[cid, pl.ds(sid * RPT, RPT)])

    return kern(x, src, dst, e)


def _edge_lin(ea, w_t, b):
    d = w_t.shape[1]
    blk = 2048

    def body(ea_ref, w_ref, b_ref, o_ref):
        o_ref[...] = jnp.dot(ea_ref[...], w_ref[...],
                             preferred_element_type=jnp.float32) + b_ref[...]

    return pl.pallas_call(
        body,
        grid=(E_PAD // blk,),
        in_specs=[pl.BlockSpec((blk, D_EDGE), lambda i: (i, 0)),
                  pl.BlockSpec((D_EDGE, d), lambda i: (0, 0)),
                  pl.BlockSpec((1, d), lambda i: (0, 0))],
        out_specs=pl.BlockSpec((blk, d), lambda i: (i, 0)),
        out_shape=jax.ShapeDtypeStruct((E_PAD, d), jnp.float32),
    )(ea, w_t, b.reshape(1, d))


def _bn(z, g, b):
    m = jnp.mean(z, axis=0, keepdims=True)
    v = jnp.mean((z - m) ** 2, axis=0, keepdims=True)
    return (z - m) / jnp.sqrt(v + 1e-5) * g + b


def _gelu(z):
    return 0.5 * z * (1.0 + lax.erf(z / jnp.sqrt(2.0).astype(jnp.float32)))


def _mlp(x, part, eps, w1_t, b1, g1, bb1, w2_t, b2, g2, bb2, w3_t=None, b3=None):
    d_mid = w1_t.shape[1]
    d_out = w2_t.shape[1]
    d_fin = d_out if w3_t is None else w3_t.shape[1]
    have3 = w3_t is not None

    def body(*refs):
        if have3:
            (x_ref, p_ref, eps_ref, w1_ref, b1_ref, g1_ref, bb1_ref,
             w2_ref, b2_ref, g2_ref, bb2_ref, w3_ref, b3_ref, o_ref) = refs
        else:
            (x_ref, p_ref, eps_ref, w1_ref, b1_ref, g1_ref, bb1_ref,
             w2_ref, b2_ref, g2_ref, bb2_ref, o_ref) = refs
        h0 = ((1.0 + eps_ref[0]) * x_ref[...]
              + p_ref[0, :N_NODES, :] + p_ref[1, :N_NODES, :])
        z = jnp.dot(h0, w1_ref[...], preferred_element_type=jnp.float32) + b1_ref[...]
        z = _gelu(_bn(z, g1_ref[...], bb1_ref[...]))
        z = jnp.dot(z, w2_ref[...], preferred_element_type=jnp.float32) + b2_ref[...]
        z = _gelu(_bn(z, g2_ref[...], bb2_ref[...]))
        if have3:
            z = jnp.dot(z, w3_ref[...], preferred_element_type=jnp.float32) + b3_ref[...]
        o_ref[...] = z

    args = [x, part, eps.reshape(1),
            w1_t, b1.reshape(1, d_mid), g1.reshape(1, d_mid), bb1.reshape(1, d_mid),
            w2_t, b2.reshape(1, d_out), g2.reshape(1, d_out), bb2.reshape(1, d_out)]
    if have3:
        args += [w3_t, b3.reshape(1, d_fin)]
    in_specs = [pl.BlockSpec(a.shape, _zero_idx(a.ndim)) for a in args]
    in_specs[2] = pl.BlockSpec(memory_space=pltpu.SMEM)
    return pl.pallas_call(
        body,
        in_specs=in_specs,
        out_specs=pl.BlockSpec((N_NODES, d_fin), _zero_idx(2)),
        out_shape=jax.ShapeDtypeStruct((N_NODES, d_fin), jnp.float32),
    )(*args)


def _zero_idx(ndim):
    return lambda *_: (0,) * ndim


def _heads(ha, hv, aw_ts, abs_, cw_t, cb):
    def body(ha_ref, hv_ref, w0, w1, w2, b0, b1, b2, cw_ref, cb_ref,
             act_ref, lp_ref, ent_ref, val_ref):
        h = ha_ref[...]
        lp_acc = jnp.zeros((N_NODES,), jnp.float32)
        ent_acc = jnp.zeros((N_NODES,), jnp.float32)
        for j, (w_ref, b_ref) in enumerate(((w0, b0), (w1, b1), (w2, b2))):
            l = jnp.dot(h, w_ref[...], preferred_element_type=jnp.float32) + b_ref[...]
            m = jnp.max(l, axis=-1, keepdims=True)
            s = l - m
            lse = jnp.log(jnp.sum(jnp.exp(s), axis=-1, keepdims=True))
            logp = s - lse
            iota = lax.broadcasted_iota(jnp.int32, l.shape, 1)
            amax = jnp.min(jnp.where(l == m, iota, l.shape[1]), axis=-1)
            act_ref[j, :] = amax
            lp_acc = lp_acc + jnp.sum(
                jnp.where(iota == amax[:, None], logp, 0.0), axis=-1)
            p = jnp.exp(logp)
            ent_acc = ent_acc - jnp.sum(p * logp, axis=-1)
        lp_ref[...] = lp_acc[:, None]
        ent_ref[...] = ent_acc[:, None]
        val_ref[...] = jnp.dot(hv_ref[...], cw_ref[...],
                               preferred_element_type=jnp.float32) + cb_ref[...]

    args = [ha, hv, *aw_ts, *[b.reshape(1, -1) for b in abs_], cw_t,
            cb.reshape(1, 1)]
    in_specs = [pl.BlockSpec(a.shape, _zero_idx(a.ndim)) for a in args]
    out_shapes = [jax.ShapeDtypeStruct((3, N_NODES), jnp.int32),
                  jax.ShapeDtypeStruct((N_NODES, 1), jnp.float32),
                  jax.ShapeDtypeStruct((N_NODES, 1), jnp.float32),
                  jax.ShapeDtypeStruct((N_NODES, 1), jnp.float32)]
    return pl.pallas_call(
        body,
        in_specs=in_specs,
        out_specs=[pl.BlockSpec(s.shape, _zero_idx(s.ndim)) for s in out_shapes],
        out_shape=out_shapes,
    )(*args)


def _latent(p, x, src, dst, ea_pad):
    e1 = _edge_lin(ea_pad, p['lin_e1_W'].T, p['lin_e1_b'])
    part1 = _gine_aggregate(x, src, dst, e1, D_NODE)
    h1 = _mlp(x, part1, p['eps1'],
              p['nn1_W1'].T, p['nn1_b1'], p['nn1_bn_g'], p['nn1_bn_b'],
              p['nn1_W2'].T, p['nn1_b2'], p['bn1_g'], p['bn1_b'])
    e2 = _edge_lin(ea_pad, p['lin_e2_W'].T, p['lin_e2_b'])
    part2 = _gine_aggregate(h1, src, dst, e2, 32)
    h2 = _mlp(h1, part2, p['eps2'],
              p['nn2_W1'].T, p['nn2_b1'], p['nn2_bn_g'], p['nn2_bn_b'],
              p['nn2_W2'].T, p['nn2_b2'], p['bn2_g'], p['bn2_b'],
              p['lin_W'].T, p['lin_b'])
    return h2


def kernel(x, edge_index, edge_attr, actor_params, value_params,
           actor_W, actor_b, critic_W, critic_b):
    pad = E_PAD - N_EDGES
    src = jnp.concatenate([edge_index[0], jnp.zeros((pad,), jnp.int32)])
    dst = jnp.concatenate([edge_index[1],
                           jnp.full((pad,), N_NODES, jnp.int32)])
    ea_pad = jnp.concatenate([edge_attr, jnp.zeros((pad, D_EDGE), jnp.float32)])

    hidden_a = _latent(actor_params, x, src, dst, ea_pad)
    hidden_v = _latent(value_params, x, src, dst, ea_pad)

    aw_ts = [actor_W[9 * j:9 * j + 9].T for j in range(3)]
    abs_ = [actor_b[9 * j:9 * j + 9] for j in range(3)]
    action, lp, ent, values = _heads(hidden_a, hidden_v, aw_ts, abs_,
                                     critic_W.T, critic_b)
    return (action, lp[:, 0], ent[:, 0], values)


# speedup vs baseline: 1.9050x; 1.9050x over previous
import functools

import jax
import jax.numpy as jnp
from jax import lax
from jax.experimental import pallas as pl
from jax.experimental.pallas import tpu as pltpu
from jax.experimental.pallas import tpu_sc as plsc

N_NODES = 10000
N_EDGES = 320000
D_NODE = 128
D_EDGE = 16

NC = 2
NSUB = 16
NW = NC * NSUB
CHUNK = 128
EPW = ((N_EDGES + NW * CHUNK - 1) // (NW * CHUNK)) * CHUNK
E_PAD = NW * EPW
N_PAD = 10240
RPT = N_PAD // NSUB
ZROWS = 64


def _gine_aggregate(x, src, dst, e, d):
    mesh = plsc.VectorSubcoreMesh(core_axis_name="c", subcore_axis_name="s")

    @functools.partial(
        pl.kernel,
        out_type=jax.ShapeDtypeStruct((NC, N_PAD, d), jnp.float32),
        mesh=mesh,
        scratch_types=[
            pltpu.VMEM((CHUNK,), jnp.int32),
            pltpu.VMEM((CHUNK,), jnp.int32),
            pltpu.VMEM((CHUNK, d), jnp.float32),
            pltpu.VMEM((CHUNK, d), jnp.float32),
            pltpu.VMEM((ZROWS, d), jnp.float32),
            pltpu.VMEM_SHARED((N_PAD, d), jnp.float32),
        ],
        compiler_params=pltpu.CompilerParams(use_tc_tiling_on_sc=False),
    )
    def kern(x_hbm, src_hbm, dst_hbm, e_hbm, out_hbm,
             srcv, dstv, rows, ev, zbuf, aggr):
        cid = lax.axis_index("c")
        sid = lax.axis_index("s")
        wid = sid * NC + cid

        @pl.loop(0, ZROWS)
        def _(i):
            for j in range(d // 16):
                zbuf.at[i, pl.ds(j * 16, 16)][...] = jnp.zeros((16,), jnp.float32)

        @pl.loop(0, RPT, step=ZROWS)
        def _(r):
            pltpu.sync_copy(zbuf, aggr.at[pl.ds(sid * RPT + r, ZROWS)])

        plsc.subcore_barrier()

        @pl.loop(0, EPW, step=CHUNK)
        def _(off):
            base = wid * EPW + off
            pltpu.sync_copy(src_hbm.at[pl.ds(base, CHUNK)], srcv)
            pltpu.sync_copy(dst_hbm.at[pl.ds(base, CHUNK)], dstv)
            pltpu.sync_copy(x_hbm.at[srcv], rows)
            pltpu.sync_copy(e_hbm.at[pl.ds(base, CHUNK)], ev)

            @pl.loop(0, CHUNK)
            def _(i):
                for j in range(d // 16):
                    sl = (i, pl.ds(j * 16, 16))
                    ev.at[*sl][...] = jnp.maximum(
                        ev.at[*sl][...] + rows.at[*sl][...], 0.0)

            pltpu.sync_copy(ev, aggr.at[dstv], add=True)

        plsc.subcore_barrier()
        pltpu.sync_copy(aggr.at[pl.ds(sid * RPT, RPT)],
                        out_hbm.at[cid, pl.ds(sid * RPT, RPT)])

    return kern(x, src, dst, e)


def _edge_lin(ea, w_t, b):
    d = w_t.shape[1]
    blk = 2048

    def body(ea_ref, w_ref, b_ref, o_ref):
        o_ref[...] = jnp.dot(ea_ref[...], w_ref[...],
                             preferred_element_type=jnp.float32, precision=lax.Precision.DEFAULT) + b_ref[...]

    return pl.pallas_call(
        body,
        grid=(E_PAD // blk,),
        in_specs=[pl.BlockSpec((blk, D_EDGE), lambda i: (i, 0)),
                  pl.BlockSpec((D_EDGE, d), lambda i: (0, 0)),
                  pl.BlockSpec((1, d), lambda i: (0, 0))],
        out_specs=pl.BlockSpec((blk, d), lambda i: (i, 0)),
        out_shape=jax.ShapeDtypeStruct((E_PAD, d), jnp.float32),
    )(ea, w_t, b.reshape(1, d))


def _bn(z, g, b):
    m = jnp.mean(z, axis=0, keepdims=True)
    v = jnp.mean((z - m) ** 2, axis=0, keepdims=True)
    return (z - m) / jnp.sqrt(v + 1e-5) * g + b


def _gelu(z):
    sqrt_half = jnp.float32(0.7071067811865476)
    return 0.5 * z * (1.0 + lax.erf(z * sqrt_half))


def _mlp(x, part, eps, w1_t, b1, g1, bb1, w2_t, b2, g2, bb2, w3_t=None, b3=None):
    d_mid = w1_t.shape[1]
    d_out = w2_t.shape[1]
    d_fin = d_out if w3_t is None else w3_t.shape[1]
    have3 = w3_t is not None

    def body(*refs):
        if have3:
            (x_ref, p_ref, eps_ref, w1_ref, b1_ref, g1_ref, bb1_ref,
             w2_ref, b2_ref, g2_ref, bb2_ref, w3_ref, b3_ref, o_ref) = refs
        else:
            (x_ref, p_ref, eps_ref, w1_ref, b1_ref, g1_ref, bb1_ref,
             w2_ref, b2_ref, g2_ref, bb2_ref, o_ref) = refs
        h0 = ((1.0 + eps_ref[0]) * x_ref[...]
              + p_ref[0, :N_NODES, :] + p_ref[1, :N_NODES, :])
        z = jnp.dot(h0, w1_ref[...], preferred_element_type=jnp.float32, precision=lax.Precision.DEFAULT) + b1_ref[...]
        z = _gelu(_bn(z, g1_ref[...], bb1_ref[...]))
        z = jnp.dot(z, w2_ref[...], preferred_element_type=jnp.float32, precision=lax.Precision.DEFAULT) + b2_ref[...]
        z = _gelu(_bn(z, g2_ref[...], bb2_ref[...]))
        if have3:
            z = jnp.dot(z, w3_ref[...], preferred_element_type=jnp.float32, precision=lax.Precision.DEFAULT) + b3_ref[...]
        o_ref[...] = z

    args = [x, part, eps.reshape(1),
            w1_t, b1.reshape(1, d_mid), g1.reshape(1, d_mid), bb1.reshape(1, d_mid),
            w2_t, b2.reshape(1, d_out), g2.reshape(1, d_out), bb2.reshape(1, d_out)]
    if have3:
        args += [w3_t, b3.reshape(1, d_fin)]
    in_specs = [pl.BlockSpec(a.shape, _zero_idx(a.ndim)) for a in args]
    in_specs[2] = pl.BlockSpec(memory_space=pltpu.SMEM)
    return pl.pallas_call(
        body,
        in_specs=in_specs,
        out_specs=pl.BlockSpec((N_NODES, d_fin), _zero_idx(2)),
        out_shape=jax.ShapeDtypeStruct((N_NODES, d_fin), jnp.float32),
    )(*args)


def _zero_idx(ndim):
    return lambda *_: (0,) * ndim


def _heads(ha, hv, aw_ts, abs_, cw_t, cb):
    bh = 2000

    def body(ha_ref, hv_ref, w0, w1, w2, b0, b1, b2, cw_ref, cb_ref,
             a0_ref, a1_ref, a2_ref, lp_ref, ent_ref, val_ref):
        h = ha_ref[...]
        lp_acc = jnp.zeros((bh,), jnp.float32)
        ent_acc = jnp.zeros((bh,), jnp.float32)
        for a_ref, w_ref, b_ref in ((a0_ref, w0, b0), (a1_ref, w1, b1),
                                    (a2_ref, w2, b2)):
            l = jnp.dot(h, w_ref[...], preferred_element_type=jnp.float32, precision=lax.Precision.DEFAULT) + b_ref[...]
            m = jnp.max(l, axis=-1, keepdims=True)
            s = l - m
            lse = jnp.log(jnp.sum(jnp.exp(s), axis=-1, keepdims=True))
            logp = s - lse
            iota = lax.broadcasted_iota(jnp.int32, l.shape, 1)
            amax = jnp.min(jnp.where(l == m, iota, l.shape[1]), axis=-1)
            a_ref[...] = amax[None, None, :]
            lp_acc = lp_acc + jnp.sum(
                jnp.where(iota == amax[:, None], logp, 0.0), axis=-1)
            p = jnp.exp(logp)
            ent_acc = ent_acc - jnp.sum(p * logp, axis=-1)
        lp_ref[...] = lp_acc[None, None, :]
        ent_ref[...] = ent_acc[None, None, :]
        val = jnp.dot(hv_ref[...], cw_ref[...],
                      preferred_element_type=jnp.float32, precision=lax.Precision.DEFAULT) + cb_ref[...]
        val_ref[...] = val[:, 0][None, None, :]

    nb = N_NODES // bh
    args = [ha, hv, *aw_ts, *[b.reshape(1, -1) for b in abs_], cw_t,
            cb.reshape(1, 1)]
    in_specs = ([pl.BlockSpec((bh, 64), lambda i: (i, 0))] * 2
                + [pl.BlockSpec(a.shape, _zero_idx(a.ndim)) for a in args[2:]])
    out_shapes = [jax.ShapeDtypeStruct((nb, 1, bh), jnp.int32)] * 3 + \
                 [jax.ShapeDtypeStruct((nb, 1, bh), jnp.float32)] * 3
    return pl.pallas_call(
        body,
        grid=(nb,),
        in_specs=in_specs,
        out_specs=[pl.BlockSpec((1, 1, bh), lambda i: (i, 0, 0))] * 6,
        out_shape=out_shapes,
    )(*args)


def _latent(p, x, src, dst, ea_pad):
    e1 = _edge_lin(ea_pad, p['lin_e1_W'].T, p['lin_e1_b'])
    part1 = _gine_aggregate(x, src, dst, e1, D_NODE)
    h1 = _mlp(x, part1, p['eps1'],
              p['nn1_W1'].T, p['nn1_b1'], p['nn1_bn_g'], p['nn1_bn_b'],
              p['nn1_W2'].T, p['nn1_b2'], p['bn1_g'], p['bn1_b'])
    e2 = _edge_lin(ea_pad, p['lin_e2_W'].T, p['lin_e2_b'])
    part2 = _gine_aggregate(h1, src, dst, e2, 32)
    h2 = _mlp(h1, part2, p['eps2'],
              p['nn2_W1'].T, p['nn2_b1'], p['nn2_bn_g'], p['nn2_bn_b'],
              p['nn2_W2'].T, p['nn2_b2'], p['bn2_g'], p['bn2_b'],
              p['lin_W'].T, p['lin_b'])
    return h2


def kernel(x, edge_index, edge_attr, actor_params, value_params,
           actor_W, actor_b, critic_W, critic_b):
    pad = E_PAD - N_EDGES
    src = jnp.concatenate([edge_index[0], jnp.zeros((pad,), jnp.int32)])
    dst = jnp.concatenate([edge_index[1],
                           jnp.full((pad,), N_NODES, jnp.int32)])
    ea_pad = jnp.concatenate([edge_attr, jnp.zeros((pad, D_EDGE), jnp.float32)])

    perm = jnp.argsort(dst, stable=True)
    src = src[perm]
    dst = dst[perm]
    ea_pad = ea_pad[perm]

    hidden_a = _latent(actor_params, x, src, dst, ea_pad)
    hidden_v = _latent(value_params, x, src, dst, ea_pad)

    aw_ts = [actor_W[9 * j:9 * j + 9].T for j in range(3)]
    abs_ = [actor_b[9 * j:9 * j + 9] for j in range(3)]
    a0, a1, a2, lp, ent, val = _heads(hidden_a, hidden_v, aw_ts, abs_,
                                      critic_W.T, critic_b)
    action = jnp.stack([a0.reshape(N_NODES), a1.reshape(N_NODES),
                        a2.reshape(N_NODES)])
    return (action, lp.reshape(N_NODES), ent.reshape(N_NODES),
            val.reshape(N_NODES, 1))
